# baseline (speedup 1.0000x reference)
import jax
import jax.numpy as jnp
from jax.experimental import pallas as pl
from jax.experimental.pallas import tpu as pltpu

EMBED_DIM = 768
DEPTH = 12
NUM_HEADS = 12
HEAD_DIM = EMBED_DIM // NUM_HEADS
PATCH = 16
IMG = 224
PGRID = IMG // PATCH
NPATCH = PGRID * PGRID
T_REAL = NPATCH + 1
T_PAD = 208
BT = 4
ROWS = BT * T_PAD
TOTAL_EXPERTS = 18
TOP_K = 3
MLP_DIM = 4 * EMBED_DIM
WCOL = 3072


def _layernorm(x, g, b):
    m = jnp.mean(x, axis=-1, keepdims=True)
    v = jnp.mean((x - m) ** 2, axis=-1, keepdims=True)
    return (x - m) / jnp.sqrt(v + 1e-6) * g + b


def _embed_body(p_ref, pw_ref, add_ref, x0_ref):
    x0_ref[...] = (
        jnp.dot(p_ref[...], pw_ref[...], preferred_element_type=jnp.float32)
        + add_ref[...]
    )


def _encoder_body(x0_ref, mask_ref, w_ref,
                  ln1g_ref, ln1b_ref, bqkv_ref, bo_ref,
                  ln2g_ref, ln2b_ref, b1_ref, b2_ref,
                  out_ref, x_ref, mid_ref):
    i = pl.program_id(0)
    j = pl.program_id(1)

    @pl.when(jnp.logical_and(i == 0, j == 0))
    def _():
        x_ref[...] = x0_ref[...]

    @pl.when(j == 0)
    def _attn():
        x = x_ref[...]
        w = w_ref[0, 0]
        h = _layernorm(x, ln1g_ref[0, 0, :], ln1b_ref[0, 0, :])
        qkv = (
            jnp.dot(h, w[:, :3 * EMBED_DIM], preferred_element_type=jnp.float32)
            + bqkv_ref[0]
        )
        mask = mask_ref[...]
        o_heads = []
        for hd in range(NUM_HEADS):
            q = qkv[:, hd * HEAD_DIM:(hd + 1) * HEAD_DIM]
            k = qkv[:, EMBED_DIM + hd * HEAD_DIM:
                    EMBED_DIM + (hd + 1) * HEAD_DIM]
            v = qkv[:, 2 * EMBED_DIM + hd * HEAD_DIM:
                    2 * EMBED_DIM + (hd + 1) * HEAD_DIM]
            s = jax.lax.dot_general(
                q, k, (((1,), (1,)), ((), ())),
                preferred_element_type=jnp.float32,
            ) * (1.0 / (HEAD_DIM ** 0.5)) + mask
            m = jnp.max(s, axis=1, keepdims=True)
            e = jnp.exp(s - m)
            p = e / jnp.sum(e, axis=1, keepdims=True)
            o_heads.append(jnp.dot(p, v, preferred_element_type=jnp.float32))
        o = jnp.concatenate(o_heads, axis=1)
        x_ref[...] = x + (
            jnp.dot(o, w[:, 3 * EMBED_DIM:3 * EMBED_DIM + EMBED_DIM],
                    preferred_element_type=jnp.float32)
            + bo_ref[0]
        )

    @pl.when(j == 1)
    def _mlp_up():
        x = x_ref[...]
        h2 = _layernorm(x, ln2g_ref[0, 0, :], ln2b_ref[0, 0, :])
        mid_ref[...] = jax.nn.gelu(
            jnp.dot(h2, w_ref[0, 0], preferred_element_type=jnp.float32)
            + b1_ref[0]
        )

    @pl.when(j == 2)
    def _mlp_down():
        x = x_ref[...] + (
            jax.lax.dot_general(
                mid_ref[...], w_ref[0, 0], (((1,), (1,)), ((), ())),
                preferred_element_type=jnp.float32)
            + b2_ref[0]
        )
        x_ref[...] = x
        out_ref[0] = x


def _gate_body(cls_ref, w1_ref, b1_ref, w2_ref, b2_ref,
               logits_ref, idx_ref, em_ref, rw_ref):
    hidden = jnp.maximum(
        jnp.dot(cls_ref[...], w1_ref[...], preferred_element_type=jnp.float32)
        + b1_ref[...], 0.0)
    logits = (
        jnp.dot(hidden, w2_ref[...], preferred_element_type=jnp.float32)
        + b2_ref[...]
    )
    logits_ref[...] = logits
    iota = jax.lax.broadcasted_iota(jnp.int32, logits.shape, 1)
    cur = logits
    vals, idxs = [], []
    for _ in range(TOP_K):
        m = jnp.max(cur, axis=1, keepdims=True)
        idx = jnp.min(jnp.where(cur == m, iota, TOTAL_EXPERTS), axis=1,
                      keepdims=True)
        vals.append(m)
        idxs.append(idx)
        cur = jnp.where(iota == idx, -jnp.inf, cur)
    topv = jnp.concatenate(vals, axis=1)
    idx_ref[...] = jnp.concatenate(idxs, axis=1)
    e = jnp.exp(topv - jnp.max(topv, axis=1, keepdims=True))
    rw3 = e / jnp.sum(e, axis=1, keepdims=True)
    em = jnp.zeros_like(logits)
    rw = jnp.zeros_like(logits)
    for k in range(TOP_K):
        hit = (iota == idxs[k]).astype(jnp.float32)
        em = em + hit
        rw = rw + hit * rw3[:, k:k + 1]
    em_ref[...] = em
    rw_ref[...] = rw


def kernel(image_st, image_ae, patch_w, patch_b, cls_tok, pos_emb,
           ln1_g, ln1_b, Wqkv, bqkv, Wo, bo, ln2_g, ln2_b, W1, b1, W2, b2,
           gate_W1, gate_b1, gate_W2, gate_b2):
    f32 = jnp.float32
    B = image_st.shape[0]

    def patches(img):
        return (img.reshape(B, 3, PGRID, PATCH, PGRID, PATCH)
                .transpose(0, 2, 4, 1, 3, 5)
                .reshape(B, NPATCH, 3 * PATCH * PATCH))

    p_all = jnp.concatenate([patches(image_st), patches(image_ae)], axis=0)
    p_rows = jnp.zeros((BT, T_PAD, EMBED_DIM), f32)
    p_rows = p_rows.at[:, 1:T_REAL, :].set(p_all).reshape(ROWS, EMBED_DIM)

    add = jnp.zeros((BT, T_PAD, EMBED_DIM), f32)
    add = add.at[:, 0, :].set(cls_tok[0, 0] + pos_emb[0, 0])
    add = add.at[:, 1:T_REAL, :].set(patch_b + pos_emb[0, 1:T_REAL])
    add = add.reshape(ROWS, EMBED_DIM)

    ridx = jnp.arange(ROWS)
    bidx, tidx = ridx // T_PAD, ridx % T_PAD
    okm = (bidx[:, None] == bidx[None, :]) & (tidx[None, :] < T_REAL)
    mask = jnp.where(okm, 0.0, -1e30).astype(f32)

    w_attn = jnp.concatenate([Wqkv, Wo], axis=2)
    w_all = jnp.stack([w_attn, W1, W2.transpose(0, 2, 1)], axis=1)

    x0 = pl.pallas_call(
        _embed_body,
        in_specs=[pl.BlockSpec((ROWS, EMBED_DIM), lambda: (0, 0)),
                  pl.BlockSpec((EMBED_DIM, EMBED_DIM), lambda: (0, 0)),
                  pl.BlockSpec((ROWS, EMBED_DIM), lambda: (0, 0))],
        out_specs=pl.BlockSpec((ROWS, EMBED_DIM), lambda: (0, 0)),
        out_shape=jax.ShapeDtypeStruct((ROWS, EMBED_DIM), f32),
    )(p_rows, patch_w, add)

    ln_spec = pl.BlockSpec((1, 1, EMBED_DIM), lambda i, j: (i, 0, 0))
    en = pl.pallas_call(
        _encoder_body,
        grid=(DEPTH, 3),
        in_specs=[
            pl.BlockSpec((ROWS, EMBED_DIM), lambda i, j: (0, 0)),
            pl.BlockSpec((ROWS, ROWS), lambda i, j: (0, 0)),
            pl.BlockSpec((1, 1, EMBED_DIM, WCOL),
                         lambda i, j: (i, j, 0, 0)),
            ln_spec,
            ln_spec,
            pl.BlockSpec((1, 1, 3 * EMBED_DIM), lambda i, j: (i, 0, 0)),
            ln_spec,
            ln_spec,
            ln_spec,
            pl.BlockSpec((1, 1, MLP_DIM), lambda i, j: (i, 0, 0)),
            ln_spec,
        ],
        out_specs=pl.BlockSpec((1, ROWS, EMBED_DIM),
                               lambda i, j: (jnp.maximum(i - 2, 0), 0, 0)),
        out_shape=jax.ShapeDtypeStruct((DEPTH - 2, ROWS, EMBED_DIM), f32),
        scratch_shapes=[pltpu.VMEM((ROWS, EMBED_DIM), f32),
                        pltpu.VMEM((ROWS, MLP_DIM), f32)],
    )(
        x0, mask, w_all,
        ln1_g.reshape(DEPTH, 1, EMBED_DIM), ln1_b.reshape(DEPTH, 1, EMBED_DIM),
        bqkv.reshape(DEPTH, 1, 3 * EMBED_DIM),
        bo.reshape(DEPTH, 1, EMBED_DIM),
        ln2_g.reshape(DEPTH, 1, EMBED_DIM), ln2_b.reshape(DEPTH, 1, EMBED_DIM),
        b1.reshape(DEPTH, 1, MLP_DIM),
        b2.reshape(DEPTH, 1, EMBED_DIM),
    )

    en4 = en.reshape(DEPTH - 2, BT, T_PAD, EMBED_DIM)
    en_st = en4[:8, 0:B, :T_REAL, :]
    en_ae = en4[:8, B:2 * B, :T_REAL, :]
    cls_final = en4[9, 0:B, 0, :]

    whole = lambda a: pl.BlockSpec(a.shape, lambda: (0,) * a.ndim)
    gb1 = gate_b1.reshape(1, -1)
    gb2 = gate_b2.reshape(1, -1)
    raw_logits, top_k_indices, expert_mask, routing_weights = pl.pallas_call(
        _gate_body,
        in_specs=[whole(cls_final), whole(gate_W1), whole(gb1),
                  whole(gate_W2), whole(gb2)],
        out_specs=[
            pl.BlockSpec((B, TOTAL_EXPERTS), lambda: (0, 0)),
            pl.BlockSpec((B, TOP_K), lambda: (0, 0)),
            pl.BlockSpec((B, TOTAL_EXPERTS), lambda: (0, 0)),
            pl.BlockSpec((B, TOTAL_EXPERTS), lambda: (0, 0)),
        ],
        out_shape=[
            jax.ShapeDtypeStruct((B, TOTAL_EXPERTS), f32),
            jax.ShapeDtypeStruct((B, TOP_K), jnp.int32),
            jax.ShapeDtypeStruct((B, TOTAL_EXPERTS), f32),
            jax.ShapeDtypeStruct((B, TOTAL_EXPERTS), f32),
        ],
    )(cls_final, gate_W1, gb1, gate_W2, gb2)

    return raw_logits, top_k_indices, expert_mask, routing_weights, en_st, en_ae


# speedup vs baseline: 1.0302x; 1.0302x over previous
import jax
import jax.numpy as jnp
from jax.experimental import pallas as pl
from jax.experimental.pallas import tpu as pltpu

EMBED_DIM = 768
DEPTH = 12
NUM_HEADS = 12
HEAD_DIM = EMBED_DIM // NUM_HEADS
PATCH = 16
IMG = 224
PGRID = IMG // PATCH
NPATCH = PGRID * PGRID
T_REAL = NPATCH + 1
T_PAD = 208
BT = 4
ROWS = BT * T_PAD
TOTAL_EXPERTS = 18
TOP_K = 3
MLP_DIM = 4 * EMBED_DIM
WCOL = 3072


def _layernorm(x, g, b):
    m = jnp.mean(x, axis=-1, keepdims=True)
    v = jnp.mean((x - m) ** 2, axis=-1, keepdims=True)
    return (x - m) / jnp.sqrt(v + 1e-6) * g + b


def _embed_body(p_ref, pw_ref, add_ref, x0_ref):
    x0_ref[...] = (
        jnp.dot(p_ref[...].astype(jnp.bfloat16), pw_ref[...],
                preferred_element_type=jnp.float32)
        + add_ref[...]
    )


def _encoder_body(x0_ref, mask_ref, w_ref,
                  ln1g_ref, ln1b_ref, bqkv_ref, bo_ref,
                  ln2g_ref, ln2b_ref, b1_ref, b2_ref,
                  out_ref, x_ref, mid_ref):
    i = pl.program_id(0)
    j = pl.program_id(1)

    @pl.when(jnp.logical_and(i == 0, j == 0))
    def _():
        x_ref[...] = x0_ref[...]

    @pl.when(j == 0)
    def _attn():
        x = x_ref[...]
        w = w_ref[0, 0]
        h = _layernorm(x, ln1g_ref[0, 0, :], ln1b_ref[0, 0, :]).astype(
            jnp.bfloat16)
        qkv = (
            jnp.dot(h, w[:, :3 * EMBED_DIM], preferred_element_type=jnp.float32)
            + bqkv_ref[0]
        )
        mask = mask_ref[...]
        o_heads = []
        for hd in range(NUM_HEADS):
            q = qkv[:, hd * HEAD_DIM:(hd + 1) * HEAD_DIM].astype(jnp.bfloat16)
            k = qkv[:, EMBED_DIM + hd * HEAD_DIM:
                    EMBED_DIM + (hd + 1) * HEAD_DIM].astype(jnp.bfloat16)
            v = qkv[:, 2 * EMBED_DIM + hd * HEAD_DIM:
                    2 * EMBED_DIM + (hd + 1) * HEAD_DIM].astype(jnp.bfloat16)
            s = jax.lax.dot_general(
                q, k, (((1,), (1,)), ((), ())),
                preferred_element_type=jnp.float32,
            ) * (1.0 / (HEAD_DIM ** 0.5)) + mask
            m = jnp.max(s, axis=1, keepdims=True)
            e = jnp.exp(s - m)
            p = (e / jnp.sum(e, axis=1, keepdims=True)).astype(jnp.bfloat16)
            o_heads.append(jnp.dot(p, v, preferred_element_type=jnp.float32))
        o = jnp.concatenate(o_heads, axis=1).astype(jnp.bfloat16)
        x_ref[...] = x + (
            jnp.dot(o, w[:, 3 * EMBED_DIM:3 * EMBED_DIM + EMBED_DIM],
                    preferred_element_type=jnp.float32)
            + bo_ref[0]
        )

    @pl.when(j == 1)
    def _mlp_up():
        x = x_ref[...]
        h2 = _layernorm(x, ln2g_ref[0, 0, :], ln2b_ref[0, 0, :]).astype(
            jnp.bfloat16)
        mid_ref[...] = jax.nn.gelu(
            jnp.dot(h2, w_ref[0, 0], preferred_element_type=jnp.float32)
            + b1_ref[0]
        ).astype(jnp.bfloat16)

    @pl.when(j == 2)
    def _mlp_down():
        x = x_ref[...] + (
            jax.lax.dot_general(
                mid_ref[...], w_ref[0, 0], (((1,), (1,)), ((), ())),
                preferred_element_type=jnp.float32)
            + b2_ref[0]
        )
        x_ref[...] = x
        out_ref[0] = x


def _gate_body(cls_ref, w1_ref, b1_ref, w2_ref, b2_ref,
               logits_ref, idx_ref, em_ref, rw_ref):
    hidden = jnp.maximum(
        jnp.dot(cls_ref[...], w1_ref[...], preferred_element_type=jnp.float32)
        + b1_ref[...], 0.0)
    logits = (
        jnp.dot(hidden, w2_ref[...], preferred_element_type=jnp.float32)
        + b2_ref[...]
    )
    logits_ref[...] = logits
    iota = jax.lax.broadcasted_iota(jnp.int32, logits.shape, 1)
    cur = logits
    vals, idxs = [], []
    for _ in range(TOP_K):
        m = jnp.max(cur, axis=1, keepdims=True)
        idx = jnp.min(jnp.where(cur == m, iota, TOTAL_EXPERTS), axis=1,
                      keepdims=True)
        vals.append(m)
        idxs.append(idx)
        cur = jnp.where(iota == idx, -jnp.inf, cur)
    topv = jnp.concatenate(vals, axis=1)
    idx_ref[...] = jnp.concatenate(idxs, axis=1)
    e = jnp.exp(topv - jnp.max(topv, axis=1, keepdims=True))
    rw3 = e / jnp.sum(e, axis=1, keepdims=True)
    em = jnp.zeros_like(logits)
    rw = jnp.zeros_like(logits)
    for k in range(TOP_K):
        hit = (iota == idxs[k]).astype(jnp.float32)
        em = em + hit
        rw = rw + hit * rw3[:, k:k + 1]
    em_ref[...] = em
    rw_ref[...] = rw


def kernel(image_st, image_ae, patch_w, patch_b, cls_tok, pos_emb,
           ln1_g, ln1_b, Wqkv, bqkv, Wo, bo, ln2_g, ln2_b, W1, b1, W2, b2,
           gate_W1, gate_b1, gate_W2, gate_b2):
    f32 = jnp.float32
    B = image_st.shape[0]

    def patches(img):
        return (img.reshape(B, 3, PGRID, PATCH, PGRID, PATCH)
                .transpose(0, 2, 4, 1, 3, 5)
                .reshape(B, NPATCH, 3 * PATCH * PATCH))

    p_all = jnp.concatenate([patches(image_st), patches(image_ae)], axis=0)
    p_rows = jnp.zeros((BT, T_PAD, EMBED_DIM), f32)
    p_rows = p_rows.at[:, 1:T_REAL, :].set(p_all).reshape(ROWS, EMBED_DIM)

    add = jnp.zeros((BT, T_PAD, EMBED_DIM), f32)
    add = add.at[:, 0, :].set(cls_tok[0, 0] + pos_emb[0, 0])
    add = add.at[:, 1:T_REAL, :].set(patch_b + pos_emb[0, 1:T_REAL])
    add = add.reshape(ROWS, EMBED_DIM)

    ridx = jnp.arange(ROWS)
    bidx, tidx = ridx // T_PAD, ridx % T_PAD
    okm = (bidx[:, None] == bidx[None, :]) & (tidx[None, :] < T_REAL)
    mask = jnp.where(okm, 0.0, -1e30).astype(f32)

    w_attn = jnp.concatenate([Wqkv, Wo], axis=2)
    w_all = jnp.stack([w_attn, W1, W2.transpose(0, 2, 1)], axis=1).astype(
        jnp.bfloat16)

    x0 = pl.pallas_call(
        _embed_body,
        in_specs=[pl.BlockSpec((ROWS, EMBED_DIM), lambda: (0, 0)),
                  pl.BlockSpec((EMBED_DIM, EMBED_DIM), lambda: (0, 0)),
                  pl.BlockSpec((ROWS, EMBED_DIM), lambda: (0, 0))],
        out_specs=pl.BlockSpec((ROWS, EMBED_DIM), lambda: (0, 0)),
        out_shape=jax.ShapeDtypeStruct((ROWS, EMBED_DIM), f32),
    )(p_rows, patch_w.astype(jnp.bfloat16), add)

    ln_spec = pl.BlockSpec((1, 1, EMBED_DIM), lambda i, j: (i, 0, 0))
    en = pl.pallas_call(
        _encoder_body,
        grid=(DEPTH, 3),
        in_specs=[
            pl.BlockSpec((ROWS, EMBED_DIM), lambda i, j: (0, 0)),
            pl.BlockSpec((ROWS, ROWS), lambda i, j: (0, 0)),
            pl.BlockSpec((1, 1, EMBED_DIM, WCOL),
                         lambda i, j: (i, j, 0, 0)),
            ln_spec,
            ln_spec,
            pl.BlockSpec((1, 1, 3 * EMBED_DIM), lambda i, j: (i, 0, 0)),
            ln_spec,
            ln_spec,
            ln_spec,
            pl.BlockSpec((1, 1, MLP_DIM), lambda i, j: (i, 0, 0)),
            ln_spec,
        ],
        out_specs=pl.BlockSpec((1, ROWS, EMBED_DIM),
                               lambda i, j: (jnp.maximum(i - 2, 0), 0, 0)),
        out_shape=jax.ShapeDtypeStruct((DEPTH - 2, ROWS, EMBED_DIM), f32),
        scratch_shapes=[pltpu.VMEM((ROWS, EMBED_DIM), f32),
                        pltpu.VMEM((ROWS, MLP_DIM), jnp.bfloat16)],
    )(
        x0, mask, w_all,
        ln1_g.reshape(DEPTH, 1, EMBED_DIM), ln1_b.reshape(DEPTH, 1, EMBED_DIM),
        bqkv.reshape(DEPTH, 1, 3 * EMBED_DIM),
        bo.reshape(DEPTH, 1, EMBED_DIM),
        ln2_g.reshape(DEPTH, 1, EMBED_DIM), ln2_b.reshape(DEPTH, 1, EMBED_DIM),
        b1.reshape(DEPTH, 1, MLP_DIM),
        b2.reshape(DEPTH, 1, EMBED_DIM),
    )

    en4 = en.reshape(DEPTH - 2, BT, T_PAD, EMBED_DIM)
    en_st = en4[:8, 0:B, :T_REAL, :]
    en_ae = en4[:8, B:2 * B, :T_REAL, :]
    cls_final = en4[9, 0:B, 0, :]

    whole = lambda a: pl.BlockSpec(a.shape, lambda: (0,) * a.ndim)
    gb1 = gate_b1.reshape(1, -1)
    gb2 = gate_b2.reshape(1, -1)
    raw_logits, top_k_indices, expert_mask, routing_weights = pl.pallas_call(
        _gate_body,
        in_specs=[whole(cls_final), whole(gate_W1), whole(gb1),
                  whole(gate_W2), whole(gb2)],
        out_specs=[
            pl.BlockSpec((B, TOTAL_EXPERTS), lambda: (0, 0)),
            pl.BlockSpec((B, TOP_K), lambda: (0, 0)),
            pl.BlockSpec((B, TOTAL_EXPERTS), lambda: (0, 0)),
            pl.BlockSpec((B, TOTAL_EXPERTS), lambda: (0, 0)),
        ],
        out_shape=[
            jax.ShapeDtypeStruct((B, TOTAL_EXPERTS), f32),
            jax.ShapeDtypeStruct((B, TOP_K), jnp.int32),
            jax.ShapeDtypeStruct((B, TOTAL_EXPERTS), f32),
            jax.ShapeDtypeStruct((B, TOTAL_EXPERTS), f32),
        ],
    )(cls_final, gate_W1, gb1, gate_W2, gb2)

    return raw_logits, top_k_indices, expert_mask, routing_weights, en_st, en_ae


# speedup vs baseline: 1.1957x; 1.1606x over previous
import jax
import jax.numpy as jnp
from jax.experimental import pallas as pl
from jax.experimental.pallas import tpu as pltpu

EMBED_DIM = 768
DEPTH = 12
NUM_HEADS = 12
HEAD_DIM = EMBED_DIM // NUM_HEADS
PATCH = 16
IMG = 224
PGRID = IMG // PATCH
NPATCH = PGRID * PGRID
T_REAL = NPATCH + 1
T_PAD = 208
BT = 4
ROWS = BT * T_PAD
TOTAL_EXPERTS = 18
TOP_K = 3
MLP_DIM = 4 * EMBED_DIM
WCOL = 3 * EMBED_DIM + EMBED_DIM + MLP_DIM + MLP_DIM


def _layernorm(x, g, b):
    m = jnp.mean(x, axis=-1, keepdims=True)
    v = jnp.mean((x - m) ** 2, axis=-1, keepdims=True)
    return (x - m) / jnp.sqrt(v + 1e-6) * g + b


def _embed_body(p_ref, pw_ref, add_ref, x0_ref):
    x0_ref[...] = (
        jnp.dot(p_ref[...].astype(jnp.bfloat16), pw_ref[...],
                preferred_element_type=jnp.float32)
        + add_ref[...]
    )


def _encoder_body(x0_ref, w_ref,
                  ln1g_ref, ln1b_ref, bqkv_ref, bo_ref,
                  ln2g_ref, ln2b_ref, b1_ref, b2_ref,
                  out_ref, x_ref):
    i = pl.program_id(0)
    bf16 = jnp.bfloat16

    @pl.when(i == 0)
    def _():
        x_ref[...] = x0_ref[...]

    x = x_ref[...]
    w = w_ref[0]

    h = _layernorm(x, ln1g_ref[0, 0, :], ln1b_ref[0, 0, :]).astype(bf16)
    qkv = (
        jnp.dot(h, w[:, :3 * EMBED_DIM], preferred_element_type=jnp.float32)
        + bqkv_ref[0]
    )
    kmask = jax.lax.broadcasted_iota(jnp.int32, (T_PAD, T_PAD), 1) < T_REAL
    o_rows = []
    for b in range(BT):
        qkv_b = qkv[b * T_PAD:(b + 1) * T_PAD]
        o_heads = []
        for hd in range(NUM_HEADS):
            q = qkv_b[:, hd * HEAD_DIM:(hd + 1) * HEAD_DIM].astype(bf16)
            k = qkv_b[:, EMBED_DIM + hd * HEAD_DIM:
                      EMBED_DIM + (hd + 1) * HEAD_DIM].astype(bf16)
            v = qkv_b[:, 2 * EMBED_DIM + hd * HEAD_DIM:
                      2 * EMBED_DIM + (hd + 1) * HEAD_DIM].astype(bf16)
            s = jax.lax.dot_general(
                q, k, (((1,), (1,)), ((), ())),
                preferred_element_type=jnp.float32,
            ) * (1.0 / (HEAD_DIM ** 0.5))
            s = jnp.where(kmask, s, -1e30)
            m = jnp.max(s, axis=1, keepdims=True)
            e = jnp.exp(s - m)
            p = (e * (1.0 / jnp.sum(e, axis=1, keepdims=True))).astype(bf16)
            o_heads.append(jnp.dot(p, v, preferred_element_type=jnp.float32))
        o_rows.append(jnp.concatenate(o_heads, axis=1))
    o = jnp.concatenate(o_rows, axis=0).astype(bf16)
    x = x + (
        jnp.dot(o, w[:, 3 * EMBED_DIM:4 * EMBED_DIM],
                preferred_element_type=jnp.float32)
        + bo_ref[0]
    )

    h2 = _layernorm(x, ln2g_ref[0, 0, :], ln2b_ref[0, 0, :]).astype(bf16)
    mid = jax.nn.gelu(
        jnp.dot(h2, w[:, 4 * EMBED_DIM:4 * EMBED_DIM + MLP_DIM],
                preferred_element_type=jnp.float32)
        + b1_ref[0]
    ).astype(bf16)
    x = x + (
        jax.lax.dot_general(
            mid, w[:, 4 * EMBED_DIM + MLP_DIM:], (((1,), (1,)), ((), ())),
            preferred_element_type=jnp.float32)
        + b2_ref[0]
    )
    x_ref[...] = x
    out_ref[0] = x


def _gate_body(cls_ref, w1_ref, b1_ref, w2_ref, b2_ref,
               logits_ref, idx_ref, em_ref, rw_ref):
    hidden = jnp.maximum(
        jnp.dot(cls_ref[...], w1_ref[...], preferred_element_type=jnp.float32)
        + b1_ref[...], 0.0)
    logits = (
        jnp.dot(hidden, w2_ref[...], preferred_element_type=jnp.float32)
        + b2_ref[...]
    )
    logits_ref[...] = logits
    iota = jax.lax.broadcasted_iota(jnp.int32, logits.shape, 1)
    cur = logits
    vals, idxs = [], []
    for _ in range(TOP_K):
        m = jnp.max(cur, axis=1, keepdims=True)
        idx = jnp.min(jnp.where(cur == m, iota, TOTAL_EXPERTS), axis=1,
                      keepdims=True)
        vals.append(m)
        idxs.append(idx)
        cur = jnp.where(iota == idx, -jnp.inf, cur)
    topv = jnp.concatenate(vals, axis=1)
    idx_ref[...] = jnp.concatenate(idxs, axis=1)
    e = jnp.exp(topv - jnp.max(topv, axis=1, keepdims=True))
    rw3 = e / jnp.sum(e, axis=1, keepdims=True)
    em = jnp.zeros_like(logits)
    rw = jnp.zeros_like(logits)
    for k in range(TOP_K):
        hit = (iota == idxs[k]).astype(jnp.float32)
        em = em + hit
        rw = rw + hit * rw3[:, k:k + 1]
    em_ref[...] = em
    rw_ref[...] = rw


def kernel(image_st, image_ae, patch_w, patch_b, cls_tok, pos_emb,
           ln1_g, ln1_b, Wqkv, bqkv, Wo, bo, ln2_g, ln2_b, W1, b1, W2, b2,
           gate_W1, gate_b1, gate_W2, gate_b2):
    f32 = jnp.float32
    B = image_st.shape[0]

    def patches(img):
        return (img.reshape(B, 3, PGRID, PATCH, PGRID, PATCH)
                .transpose(0, 2, 4, 1, 3, 5)
                .reshape(B, NPATCH, 3 * PATCH * PATCH))

    p_all = jnp.concatenate([patches(image_st), patches(image_ae)], axis=0)
    p_rows = jnp.zeros((BT, T_PAD, EMBED_DIM), f32)
    p_rows = p_rows.at[:, 1:T_REAL, :].set(p_all).reshape(ROWS, EMBED_DIM)

    add = jnp.zeros((BT, T_PAD, EMBED_DIM), f32)
    add = add.at[:, 0, :].set(cls_tok[0, 0] + pos_emb[0, 0])
    add = add.at[:, 1:T_REAL, :].set(patch_b + pos_emb[0, 1:T_REAL])
    add = add.reshape(ROWS, EMBED_DIM)

    w_all = jnp.concatenate(
        [Wqkv, Wo, W1, W2.transpose(0, 2, 1)], axis=2).astype(jnp.bfloat16)

    x0 = pl.pallas_call(
        _embed_body,
        in_specs=[pl.BlockSpec((ROWS, EMBED_DIM), lambda: (0, 0)),
                  pl.BlockSpec((EMBED_DIM, EMBED_DIM), lambda: (0, 0)),
                  pl.BlockSpec((ROWS, EMBED_DIM), lambda: (0, 0))],
        out_specs=pl.BlockSpec((ROWS, EMBED_DIM), lambda: (0, 0)),
        out_shape=jax.ShapeDtypeStruct((ROWS, EMBED_DIM), f32),
    )(p_rows, patch_w.astype(jnp.bfloat16), add)

    ln_spec = pl.BlockSpec((1, 1, EMBED_DIM), lambda i: (i, 0, 0))
    en = pl.pallas_call(
        _encoder_body,
        grid=(DEPTH,),
        in_specs=[
            pl.BlockSpec((ROWS, EMBED_DIM), lambda i: (0, 0)),
            pl.BlockSpec((1, EMBED_DIM, WCOL), lambda i: (i, 0, 0)),
            ln_spec,
            ln_spec,
            pl.BlockSpec((1, 1, 3 * EMBED_DIM), lambda i: (i, 0, 0)),
            ln_spec,
            ln_spec,
            ln_spec,
            pl.BlockSpec((1, 1, MLP_DIM), lambda i: (i, 0, 0)),
            ln_spec,
        ],
        out_specs=pl.BlockSpec((1, ROWS, EMBED_DIM),
                               lambda i: (jnp.maximum(i - 2, 0), 0, 0)),
        out_shape=jax.ShapeDtypeStruct((DEPTH - 2, ROWS, EMBED_DIM), f32),
        scratch_shapes=[pltpu.VMEM((ROWS, EMBED_DIM), f32)],
    )(
        x0, w_all,
        ln1_g.reshape(DEPTH, 1, EMBED_DIM), ln1_b.reshape(DEPTH, 1, EMBED_DIM),
        bqkv.reshape(DEPTH, 1, 3 * EMBED_DIM),
        bo.reshape(DEPTH, 1, EMBED_DIM),
        ln2_g.reshape(DEPTH, 1, EMBED_DIM), ln2_b.reshape(DEPTH, 1, EMBED_DIM),
        b1.reshape(DEPTH, 1, MLP_DIM),
        b2.reshape(DEPTH, 1, EMBED_DIM),
    )

    en4 = en.reshape(DEPTH - 2, BT, T_PAD, EMBED_DIM)
    en_st = en4[:8, 0:B, :T_REAL, :]
    en_ae = en4[:8, B:2 * B, :T_REAL, :]
    cls_final = en4[9, 0:B, 0, :]

    whole = lambda a: pl.BlockSpec(a.shape, lambda: (0,) * a.ndim)
    gb1 = gate_b1.reshape(1, -1)
    gb2 = gate_b2.reshape(1, -1)
    raw_logits, top_k_indices, expert_mask, routing_weights = pl.pallas_call(
        _gate_body,
        in_specs=[whole(cls_final), whole(gate_W1), whole(gb1),
                  whole(gate_W2), whole(gb2)],
        out_specs=[
            pl.BlockSpec((B, TOTAL_EXPERTS), lambda: (0, 0)),
            pl.BlockSpec((B, TOP_K), lambda: (0, 0)),
            pl.BlockSpec((B, TOTAL_EXPERTS), lambda: (0, 0)),
            pl.BlockSpec((B, TOTAL_EXPERTS), lambda: (0, 0)),
        ],
        out_shape=[
            jax.ShapeDtypeStruct((B, TOTAL_EXPERTS), f32),
            jax.ShapeDtypeStruct((B, TOP_K), jnp.int32),
            jax.ShapeDtypeStruct((B, TOTAL_EXPERTS), f32),
            jax.ShapeDtypeStruct((B, TOTAL_EXPERTS), f32),
        ],
    )(cls_final, gate_W1, gb1, gate_W2, gb2)

    return raw_logits, top_k_indices, expert_mask, routing_weights, en_st, en_ae


# speedup vs baseline: 1.5126x; 1.2650x over previous
import jax
import jax.numpy as jnp
from jax.experimental import pallas as pl
from jax.experimental.pallas import tpu as pltpu

EMBED_DIM = 768
DEPTH = 12
NUM_HEADS = 12
HEAD_DIM = EMBED_DIM // NUM_HEADS
PATCH = 16
IMG = 224
PGRID = IMG // PATCH
NPATCH = PGRID * PGRID
T_REAL = NPATCH + 1
T_PAD = 208
BT = 4
ROWS = BT * T_PAD
TOTAL_EXPERTS = 18
TOP_K = 3
MLP_DIM = 4 * EMBED_DIM
N_EN = 8


def _layernorm(x, g, b):
    m = jnp.mean(x, axis=-1, keepdims=True)
    v = jnp.mean((x - m) ** 2, axis=-1, keepdims=True)
    return (x - m) / jnp.sqrt(v + 1e-6) * g + b


def _layer(x, nb, wqkv_ref, wo_ref, w1_ref, w2_ref,
           ln1g_ref, ln1b_ref, bqkv_ref, bo_ref,
           ln2g_ref, ln2b_ref, b1_ref, b2_ref):
    bf16 = jnp.bfloat16
    h = _layernorm(x, ln1g_ref[0, 0, :], ln1b_ref[0, 0, :]).astype(bf16)
    qkv = (
        jnp.dot(h, wqkv_ref[0], preferred_element_type=jnp.float32)
        + bqkv_ref[0]
    )
    kmask = jax.lax.broadcasted_iota(jnp.int32, (T_PAD, T_PAD), 1) < T_REAL
    o_rows = []
    for b in range(nb):
        qkv_b = qkv[b * T_PAD:(b + 1) * T_PAD]
        o_heads = []
        for hd in range(NUM_HEADS):
            q = qkv_b[:, hd * HEAD_DIM:(hd + 1) * HEAD_DIM].astype(bf16)
            k = qkv_b[:, EMBED_DIM + hd * HEAD_DIM:
                      EMBED_DIM + (hd + 1) * HEAD_DIM].astype(bf16)
            v = qkv_b[:, 2 * EMBED_DIM + hd * HEAD_DIM:
                      2 * EMBED_DIM + (hd + 1) * HEAD_DIM].astype(bf16)
            s = jax.lax.dot_general(
                q, k, (((1,), (1,)), ((), ())),
                preferred_element_type=jnp.float32,
            ) * (1.0 / (HEAD_DIM ** 0.5))
            s = jnp.where(kmask, s, -1e30)
            m = jnp.max(s, axis=1, keepdims=True)
            e = jnp.exp(s - m)
            p = (e * (1.0 / jnp.sum(e, axis=1, keepdims=True))).astype(bf16)
            o_heads.append(jnp.dot(p, v, preferred_element_type=jnp.float32))
        o_rows.append(jnp.concatenate(o_heads, axis=1))
    o = jnp.concatenate(o_rows, axis=0).astype(bf16)
    x = x + (
        jnp.dot(o, wo_ref[0], preferred_element_type=jnp.float32)
        + bo_ref[0]
    )
    h2 = _layernorm(x, ln2g_ref[0, 0, :], ln2b_ref[0, 0, :]).astype(bf16)
    mid = jax.nn.gelu(
        jnp.dot(h2, w1_ref[0], preferred_element_type=jnp.float32)
        + b1_ref[0]
    ).astype(bf16)
    return x + (
        jnp.dot(mid, w2_ref[0], preferred_element_type=jnp.float32)
        + b2_ref[0]
    )


def _fwd_body(p_ref, pw_ref, add_ref,
              wqkv_ref, wo_ref, w1_ref, w2_ref,
              ln1g_ref, ln1b_ref, bqkv_ref, bo_ref,
              ln2g_ref, ln2b_ref, b1_ref, b2_ref,
              gw1_ref, gb1_ref, gw2_ref, gb2_ref,
              st_ref, ae_ref, logits_ref, idx_ref, em_ref, rw_ref,
              x_ref):
    i = pl.program_id(0)

    @pl.when(i == 0)
    def _():
        x_ref[...] = (
            jnp.dot(p_ref[...], pw_ref[...],
                    preferred_element_type=jnp.float32)
            + add_ref[...]
        )

    wrefs = (wqkv_ref, wo_ref, w1_ref, w2_ref,
             ln1g_ref, ln1b_ref, bqkv_ref, bo_ref,
             ln2g_ref, ln2b_ref, b1_ref, b2_ref)

    @pl.when(i < 10)
    def _full():
        x_ref[...] = _layer(x_ref[...], BT, *wrefs)

    @pl.when(i >= 10)
    def _half():
        x_ref[0:2 * T_PAD] = _layer(x_ref[0:2 * T_PAD], 2, *wrefs)

    @pl.when(jnp.logical_and(i >= 2, i < 10))
    def _emit():
        x4 = x_ref[...].reshape(BT, T_PAD, EMBED_DIM)
        st_ref[0] = x4[0:2, 0:T_REAL, :]
        ae_ref[0] = x4[2:4, 0:T_REAL, :]

    @pl.when(i == DEPTH - 1)
    def _gate():
        x4 = x_ref[...].reshape(BT, T_PAD, EMBED_DIM)
        cls = x4[0:2, 0, :]
        hidden = jnp.maximum(
            jnp.dot(cls, gw1_ref[...], preferred_element_type=jnp.float32)
            + gb1_ref[...], 0.0)
        logits = (
            jnp.dot(hidden, gw2_ref[...], preferred_element_type=jnp.float32)
            + gb2_ref[...]
        )
        logits_ref[...] = logits
        iota = jax.lax.broadcasted_iota(jnp.int32, logits.shape, 1)
        cur = logits
        vals, idxs = [], []
        for _ in range(TOP_K):
            m = jnp.max(cur, axis=1, keepdims=True)
            idx = jnp.min(jnp.where(cur == m, iota, TOTAL_EXPERTS), axis=1,
                          keepdims=True)
            vals.append(m)
            idxs.append(idx)
            cur = jnp.where(iota == idx, -jnp.inf, cur)
        topv = jnp.concatenate(vals, axis=1)
        idx_ref[...] = jnp.concatenate(idxs, axis=1)
        e = jnp.exp(topv - jnp.max(topv, axis=1, keepdims=True))
        rw3 = e / jnp.sum(e, axis=1, keepdims=True)
        em = jnp.zeros_like(logits)
        rw = jnp.zeros_like(logits)
        for k in range(TOP_K):
            hit = (iota == idxs[k]).astype(jnp.float32)
            em = em + hit
            rw = rw + hit * rw3[:, k:k + 1]
        em_ref[...] = em
        rw_ref[...] = rw


def kernel(image_st, image_ae, patch_w, patch_b, cls_tok, pos_emb,
           ln1_g, ln1_b, Wqkv, bqkv, Wo, bo, ln2_g, ln2_b, W1, b1, W2, b2,
           gate_W1, gate_b1, gate_W2, gate_b2):
    f32 = jnp.float32
    bf16 = jnp.bfloat16
    B = image_st.shape[0]

    def patches(img):
        return (img.reshape(B, 3, PGRID, PATCH, PGRID, PATCH)
                .transpose(0, 2, 4, 1, 3, 5)
                .reshape(B, NPATCH, 3 * PATCH * PATCH))

    p_all = jnp.concatenate([patches(image_st), patches(image_ae)], axis=0)
    p_rows = jnp.zeros((BT, T_PAD, EMBED_DIM), bf16)
    p_rows = p_rows.at[:, 1:T_REAL, :].set(
        p_all.astype(bf16)).reshape(ROWS, EMBED_DIM)

    add = jnp.zeros((BT, T_PAD, EMBED_DIM), f32)
    add = add.at[:, 0, :].set(cls_tok[0, 0] + pos_emb[0, 0])
    add = add.at[:, 1:T_REAL, :].set(patch_b + pos_emb[0, 1:T_REAL])
    add = add.reshape(ROWS, EMBED_DIM)

    const2 = lambda shape: pl.BlockSpec(shape, lambda i: (0, 0))
    perlayer = lambda shape: pl.BlockSpec((1,) + shape,
                                          lambda i: (i,) + (0,) * len(shape))
    en_spec = pl.BlockSpec((1, B, T_REAL, EMBED_DIM),
                           lambda i: (jnp.clip(i - 2, 0, N_EN - 1), 0, 0, 0))

    outs = pl.pallas_call(
        _fwd_body,
        grid=(DEPTH,),
        in_specs=[
            const2((ROWS, EMBED_DIM)),
            const2((EMBED_DIM, EMBED_DIM)),
            const2((ROWS, EMBED_DIM)),
            perlayer((EMBED_DIM, 3 * EMBED_DIM)),
            perlayer((EMBED_DIM, EMBED_DIM)),
            perlayer((EMBED_DIM, MLP_DIM)),
            perlayer((MLP_DIM, EMBED_DIM)),
            perlayer((1, EMBED_DIM)),
            perlayer((1, EMBED_DIM)),
            perlayer((1, 3 * EMBED_DIM)),
            perlayer((1, EMBED_DIM)),
            perlayer((1, EMBED_DIM)),
            perlayer((1, EMBED_DIM)),
            perlayer((1, MLP_DIM)),
            perlayer((1, EMBED_DIM)),
            const2((EMBED_DIM, 256)),
            const2((1, 256)),
            const2((256, TOTAL_EXPERTS)),
            const2((1, TOTAL_EXPERTS)),
        ],
        out_specs=[
            en_spec,
            en_spec,
            const2((B, TOTAL_EXPERTS)),
            const2((B, TOP_K)),
            const2((B, TOTAL_EXPERTS)),
            const2((B, TOTAL_EXPERTS)),
        ],
        out_shape=[
            jax.ShapeDtypeStruct((N_EN, B, T_REAL, EMBED_DIM), f32),
            jax.ShapeDtypeStruct((N_EN, B, T_REAL, EMBED_DIM), f32),
            jax.ShapeDtypeStruct((B, TOTAL_EXPERTS), f32),
            jax.ShapeDtypeStruct((B, TOP_K), jnp.int32),
            jax.ShapeDtypeStruct((B, TOTAL_EXPERTS), f32),
            jax.ShapeDtypeStruct((B, TOTAL_EXPERTS), f32),
        ],
        scratch_shapes=[pltpu.VMEM((ROWS, EMBED_DIM), f32)],
    )(
        p_rows, patch_w.astype(bf16), add,
        Wqkv.astype(bf16), Wo.astype(bf16), W1.astype(bf16), W2.astype(bf16),
        ln1_g.reshape(DEPTH, 1, EMBED_DIM), ln1_b.reshape(DEPTH, 1, EMBED_DIM),
        bqkv.reshape(DEPTH, 1, 3 * EMBED_DIM),
        bo.reshape(DEPTH, 1, EMBED_DIM),
        ln2_g.reshape(DEPTH, 1, EMBED_DIM), ln2_b.reshape(DEPTH, 1, EMBED_DIM),
        b1.reshape(DEPTH, 1, MLP_DIM),
        b2.reshape(DEPTH, 1, EMBED_DIM),
        gate_W1, gate_b1.reshape(1, -1), gate_W2, gate_b2.reshape(1, -1),
    )
    en_st, en_ae, raw_logits, top_k_indices, expert_mask, routing_weights = outs
    return raw_logits, top_k_indices, expert_mask, routing_weights, en_st, en_ae


# speedup vs baseline: 1.5172x; 1.0031x over previous
import jax
import jax.numpy as jnp
from jax.experimental import pallas as pl
from jax.experimental.pallas import tpu as pltpu

EMBED_DIM = 768
DEPTH = 12
NUM_HEADS = 12
HEAD_DIM = EMBED_DIM // NUM_HEADS
PATCH = 16
IMG = 224
PGRID = IMG // PATCH
NPATCH = PGRID * PGRID
T_REAL = NPATCH + 1
T_PAD = 208
BT = 4
ROWS = BT * T_PAD
TOTAL_EXPERTS = 18
TOP_K = 3
MLP_DIM = 4 * EMBED_DIM
N_EN = 8


def _layernorm(x, g, b):
    m = jnp.mean(x, axis=-1, keepdims=True)
    v = jnp.mean((x - m) ** 2, axis=-1, keepdims=True)
    return (x - m) / jnp.sqrt(v + 1e-6) * g + b


def _layer(x, nb, wqkv_ref, wo_ref, w1_ref, w2_ref,
           ln1g_ref, ln1b_ref, bqkv_ref, bo_ref,
           ln2g_ref, ln2b_ref, b1_ref, b2_ref):
    bf16 = jnp.bfloat16
    h = _layernorm(x, ln1g_ref[0, 0, :], ln1b_ref[0, 0, :]).astype(bf16)
    qkv = (
        jnp.dot(h, wqkv_ref[0], preferred_element_type=jnp.float32)
        + bqkv_ref[0]
    )
    kmask = jax.lax.broadcasted_iota(jnp.int32, (T_PAD, T_PAD), 1) < T_REAL
    o_rows = []
    for b in range(nb):
        qkv_b = qkv[b * T_PAD:(b + 1) * T_PAD]
        o_heads = []
        for hd in range(NUM_HEADS):
            q = qkv_b[:, hd * HEAD_DIM:(hd + 1) * HEAD_DIM].astype(bf16)
            k = qkv_b[:, EMBED_DIM + hd * HEAD_DIM:
                      EMBED_DIM + (hd + 1) * HEAD_DIM].astype(bf16)
            v = qkv_b[:, 2 * EMBED_DIM + hd * HEAD_DIM:
                      2 * EMBED_DIM + (hd + 1) * HEAD_DIM].astype(bf16)
            s = jax.lax.dot_general(
                q, k, (((1,), (1,)), ((), ())),
                preferred_element_type=jnp.float32,
            ) * (1.0 / (HEAD_DIM ** 0.5))
            s = jnp.where(kmask, s, -1e30)
            m = jnp.max(s, axis=1, keepdims=True)
            e = jnp.exp(s - m)
            r = 1.0 / jnp.sum(e, axis=1, keepdims=True)
            o_h = jnp.dot(e.astype(bf16), v,
                          preferred_element_type=jnp.float32)
            o_heads.append(o_h * r)
        o_rows.append(jnp.concatenate(o_heads, axis=1))
    o = jnp.concatenate(o_rows, axis=0).astype(bf16)
    x = x + (
        jnp.dot(o, wo_ref[0], preferred_element_type=jnp.float32)
        + bo_ref[0]
    )
    h2 = _layernorm(x, ln2g_ref[0, 0, :], ln2b_ref[0, 0, :]).astype(bf16)
    mid = jax.nn.gelu(
        jnp.dot(h2, w1_ref[0], preferred_element_type=jnp.float32)
        + b1_ref[0]
    ).astype(bf16)
    return x + (
        jnp.dot(mid, w2_ref[0], preferred_element_type=jnp.float32)
        + b2_ref[0]
    )


def _fwd_body(p_ref, pw_ref, add_ref,
              wqkv_ref, wo_ref, w1_ref, w2_ref,
              ln1g_ref, ln1b_ref, bqkv_ref, bo_ref,
              ln2g_ref, ln2b_ref, b1_ref, b2_ref,
              gw1_ref, gb1_ref, gw2_ref, gb2_ref,
              st_ref, ae_ref, logits_ref, idx_ref, em_ref, rw_ref,
              x_ref):
    i = pl.program_id(0)

    @pl.when(i == 0)
    def _():
        x_ref[...] = (
            jnp.dot(p_ref[...], pw_ref[...],
                    preferred_element_type=jnp.float32)
            + add_ref[...]
        )

    wrefs = (wqkv_ref, wo_ref, w1_ref, w2_ref,
             ln1g_ref, ln1b_ref, bqkv_ref, bo_ref,
             ln2g_ref, ln2b_ref, b1_ref, b2_ref)

    @pl.when(i < 10)
    def _full():
        x_ref[...] = _layer(x_ref[...], BT, *wrefs)

    @pl.when(i >= 10)
    def _half():
        x_ref[0:2 * T_PAD] = _layer(x_ref[0:2 * T_PAD], 2, *wrefs)

    @pl.when(jnp.logical_and(i >= 2, i < 10))
    def _emit():
        x4 = x_ref[...].reshape(BT, T_PAD, EMBED_DIM)
        st_ref[0] = x4[0:2, 0:T_REAL, :]
        ae_ref[0] = x4[2:4, 0:T_REAL, :]

    @pl.when(i == DEPTH - 1)
    def _gate():
        x4 = x_ref[...].reshape(BT, T_PAD, EMBED_DIM)
        cls = x4[0:2, 0, :]
        hidden = jnp.maximum(
            jnp.dot(cls, gw1_ref[...], preferred_element_type=jnp.float32)
            + gb1_ref[...], 0.0)
        logits = (
            jnp.dot(hidden, gw2_ref[...], preferred_element_type=jnp.float32)
            + gb2_ref[...]
        )
        logits_ref[...] = logits
        iota = jax.lax.broadcasted_iota(jnp.int32, logits.shape, 1)
        cur = logits
        vals, idxs = [], []
        for _ in range(TOP_K):
            m = jnp.max(cur, axis=1, keepdims=True)
            idx = jnp.min(jnp.where(cur == m, iota, TOTAL_EXPERTS), axis=1,
                          keepdims=True)
            vals.append(m)
            idxs.append(idx)
            cur = jnp.where(iota == idx, -jnp.inf, cur)
        topv = jnp.concatenate(vals, axis=1)
        idx_ref[...] = jnp.concatenate(idxs, axis=1)
        e = jnp.exp(topv - jnp.max(topv, axis=1, keepdims=True))
        rw3 = e / jnp.sum(e, axis=1, keepdims=True)
        em = jnp.zeros_like(logits)
        rw = jnp.zeros_like(logits)
        for k in range(TOP_K):
            hit = (iota == idxs[k]).astype(jnp.float32)
            em = em + hit
            rw = rw + hit * rw3[:, k:k + 1]
        em_ref[...] = em
        rw_ref[...] = rw


def kernel(image_st, image_ae, patch_w, patch_b, cls_tok, pos_emb,
           ln1_g, ln1_b, Wqkv, bqkv, Wo, bo, ln2_g, ln2_b, W1, b1, W2, b2,
           gate_W1, gate_b1, gate_W2, gate_b2):
    f32 = jnp.float32
    bf16 = jnp.bfloat16
    B = image_st.shape[0]

    def patches(img):
        return (img.reshape(B, 3, PGRID, PATCH, PGRID, PATCH)
                .transpose(0, 2, 4, 1, 3, 5)
                .reshape(B, NPATCH, 3 * PATCH * PATCH))

    p_all = jnp.concatenate([patches(image_st), patches(image_ae)], axis=0)
    p_rows = jnp.zeros((BT, T_PAD, EMBED_DIM), bf16)
    p_rows = p_rows.at[:, 1:T_REAL, :].set(
        p_all.astype(bf16)).reshape(ROWS, EMBED_DIM)

    add = jnp.zeros((BT, T_PAD, EMBED_DIM), f32)
    add = add.at[:, 0, :].set(cls_tok[0, 0] + pos_emb[0, 0])
    add = add.at[:, 1:T_REAL, :].set(patch_b + pos_emb[0, 1:T_REAL])
    add = add.reshape(ROWS, EMBED_DIM)

    const2 = lambda shape: pl.BlockSpec(shape, lambda i: (0, 0))
    perlayer = lambda shape: pl.BlockSpec((1,) + shape,
                                          lambda i: (i,) + (0,) * len(shape))
    en_spec = pl.BlockSpec((1, B, T_REAL, EMBED_DIM),
                           lambda i: (jnp.clip(i - 2, 0, N_EN - 1), 0, 0, 0))

    outs = pl.pallas_call(
        _fwd_body,
        grid=(DEPTH,),
        in_specs=[
            const2((ROWS, EMBED_DIM)),
            const2((EMBED_DIM, EMBED_DIM)),
            const2((ROWS, EMBED_DIM)),
            perlayer((EMBED_DIM, 3 * EMBED_DIM)),
            perlayer((EMBED_DIM, EMBED_DIM)),
            perlayer((EMBED_DIM, MLP_DIM)),
            perlayer((MLP_DIM, EMBED_DIM)),
            perlayer((1, EMBED_DIM)),
            perlayer((1, EMBED_DIM)),
            perlayer((1, 3 * EMBED_DIM)),
            perlayer((1, EMBED_DIM)),
            perlayer((1, EMBED_DIM)),
            perlayer((1, EMBED_DIM)),
            perlayer((1, MLP_DIM)),
            perlayer((1, EMBED_DIM)),
            const2((EMBED_DIM, 256)),
            const2((1, 256)),
            const2((256, TOTAL_EXPERTS)),
            const2((1, TOTAL_EXPERTS)),
        ],
        out_specs=[
            en_spec,
            en_spec,
            const2((B, TOTAL_EXPERTS)),
            const2((B, TOP_K)),
            const2((B, TOTAL_EXPERTS)),
            const2((B, TOTAL_EXPERTS)),
        ],
        out_shape=[
            jax.ShapeDtypeStruct((N_EN, B, T_REAL, EMBED_DIM), f32),
            jax.ShapeDtypeStruct((N_EN, B, T_REAL, EMBED_DIM), f32),
            jax.ShapeDtypeStruct((B, TOTAL_EXPERTS), f32),
            jax.ShapeDtypeStruct((B, TOP_K), jnp.int32),
            jax.ShapeDtypeStruct((B, TOTAL_EXPERTS), f32),
            jax.ShapeDtypeStruct((B, TOTAL_EXPERTS), f32),
        ],
        scratch_shapes=[pltpu.VMEM((ROWS, EMBED_DIM), f32)],
    )(
        p_rows, patch_w.astype(bf16), add,
        Wqkv.astype(bf16), Wo.astype(bf16), W1.astype(bf16), W2.astype(bf16),
        ln1_g.reshape(DEPTH, 1, EMBED_DIM), ln1_b.reshape(DEPTH, 1, EMBED_DIM),
        bqkv.reshape(DEPTH, 1, 3 * EMBED_DIM),
        bo.reshape(DEPTH, 1, EMBED_DIM),
        ln2_g.reshape(DEPTH, 1, EMBED_DIM), ln2_b.reshape(DEPTH, 1, EMBED_DIM),
        b1.reshape(DEPTH, 1, MLP_DIM),
        b2.reshape(DEPTH, 1, EMBED_DIM),
        gate_W1, gate_b1.reshape(1, -1), gate_W2, gate_b2.reshape(1, -1),
    )
    en_st, en_ae, raw_logits, top_k_indices, expert_mask, routing_weights = outs
    return raw_logits, top_k_indices, expert_mask, routing_weights, en_st, en_ae
